# Initial kernel scaffold; baseline (speedup 1.0000x reference)
import jax
import jax.numpy as jnp
from jax.experimental import pallas as pl


def kernel(ins, edge_index, emb, W1, b1, W2, b2, Wc, bc):
    raise NotImplementedError("write your pallas kernel here")


# speedup vs baseline: 3.3833x; 3.3833x over previous
import functools

import jax
import jax.numpy as jnp
import numpy as np
from jax import lax
from jax.experimental import pallas as pl
from jax.experimental.pallas import tpu as pltpu
from jax.experimental.pallas import tpu_sc as plsc

N = 10000
E = 320000
V = 100000
L = 20
D = 128
H = 128
C = 10

NC = 2
NS = 16
NPAD = 10240
NPC = NPAD // NC
NPT = NPC // NS
EK = 80
EPC = E // NC
EPT = EPC // NS
APT = NPAD // NS

_mesh = plsc.VectorSubcoreMesh(core_axis_name="c", subcore_axis_name="s")


@functools.partial(
    pl.kernel,
    out_type=jax.ShapeDtypeStruct((NPAD, D), jnp.float32),
    mesh=_mesh,
    scratch_types=[
        pltpu.VMEM((EK,), jnp.int32),
        pltpu.VMEM((EK,), jnp.int32),
        pltpu.VMEM((EK, D), jnp.float32),
        pltpu.VMEM_SHARED((NPC, D), jnp.float32),
        pltpu.SemaphoreType.DMA,
    ],
)
def _sc_emb(ins_hbm, sid_hbm, emb_hbm, z128_hbm, hsum_hbm,
            gidx, sidv, rows, acc_h, sem):
    c = lax.axis_index("c")
    s = lax.axis_index("s")
    gnb = c * NPC + s * NPT

    pltpu.sync_copy(z128_hbm.at[pl.ds(0, EK)], rows)

    def zero_h(b, _):
        pltpu.sync_copy(rows, acc_h.at[pl.ds(s * NPT + b * EK, EK)])
        return _
    lax.fori_loop(0, NPT // EK, zero_h, None)
    plsc.subcore_barrier()

    def emb_chunk(k, _):
        base = gnb * L + k * EK
        pltpu.sync_copy(ins_hbm.at[pl.ds(base, EK)], gidx)
        pltpu.sync_copy(sid_hbm.at[pl.ds(base, EK)], sidv)
        pltpu.async_copy(emb_hbm.at[gidx], rows, sem).wait()
        pltpu.sync_copy(rows, acc_h.at[sidv], add=True)
        return _
    lax.fori_loop(0, (NPT * L) // EK, emb_chunk, None)
    plsc.subcore_barrier()

    def out_h(b, _):
        pltpu.sync_copy(acc_h.at[pl.ds(s * NPT + b * EK, EK)], rows)
        pltpu.sync_copy(rows, hsum_hbm.at[pl.ds(gnb + b * EK, EK)])
        return _
    lax.fori_loop(0, NPT // EK, out_h, None)


PN = NPC // 2
EPT_D = E // NS
ECH = 2000
RPT_D = PN // NS


@functools.partial(
    pl.kernel,
    out_type=jax.ShapeDtypeStruct((2 * NPAD * 16,), jnp.float32),
    mesh=_mesh,
    scratch_types=[
        pltpu.VMEM((ECH,), jnp.int32),
        pltpu.VMEM(((PN + 1) * 16,), jnp.float32),
        pltpu.VMEM((RPT_D * 16,), jnp.float32),
        pltpu.VMEM_SHARED((NS, PN * 16), jnp.float32),
    ],
    compiler_params=pltpu.CompilerParams(needs_layout_passes=False),
)
def _sc_deg(src_hbm, dst_hbm, deg_hbm, eidx, hist, out16, shp):
    c = lax.axis_index("c")
    s = lax.axis_index("s")
    lane = lax.iota(jnp.int32, 16)
    ones = jnp.full((16,), 1.0, jnp.float32)
    zeros = jnp.zeros((16,), jnp.float32)

    for p in range(NPAD // PN):
        def zero_row(r, _):
            hist[pl.ds(r * 16, 16)] = zeros
            return _
        lax.fori_loop(0, PN + 1, zero_row, None)

        def scatter(ref):
            def outer(i, _):
                pltpu.sync_copy(ref.at[pl.ds(s * EPT_D + i * ECH, ECH)], eidx)

                def inner(j, __):
                    ids = eidx[pl.ds(j * 16, 16)]
                    rel = ids - p * PN
                    ok = (rel >= 0) & (rel < PN)
                    addr = jnp.where(ok, rel, PN) * 16 + lane
                    v = plsc.load_gather(hist, [addr])
                    plsc.store_scatter(hist, [addr], v + ones)
                    return __
                lax.fori_loop(0, ECH // 16, inner, None)
                return _
            lax.fori_loop(0, EPT_D // ECH, outer, None)

        @pl.when(c == 0)
        def _():
            scatter(src_hbm)

        @pl.when(c == 1)
        def _():
            scatter(dst_hbm)

        pltpu.sync_copy(hist.at[pl.ds(0, PN * 16)], shp.at[s])
        plsc.subcore_barrier()

        def fetch(t, _):
            pltpu.sync_copy(shp.at[t, pl.ds(s * RPT_D * 16, RPT_D * 16)],
                            hist.at[pl.ds(t * RPT_D * 16, RPT_D * 16)])
            return _
        lax.fori_loop(0, NS, fetch, None)

        def reduce_row(r, _):
            v = hist[pl.ds(r * 16, 16)]
            for t in range(1, NS):
                v = v + hist[pl.ds((t * RPT_D + r) * 16, 16)]
            out16[pl.ds(r * 16, 16)] = v
            return _
        lax.fori_loop(0, RPT_D, reduce_row, None)

        pltpu.sync_copy(
            out16,
            deg_hbm.at[pl.ds((c * NPAD + p * PN + s * RPT_D) * 16,
                             RPT_D * 16)])
        plsc.subcore_barrier()


def _make_edge_agg():
    @functools.partial(
        pl.kernel,
        out_type=jax.ShapeDtypeStruct((2 * NPAD, D), jnp.float32),
        mesh=_mesh,
        scratch_types=[
            pltpu.VMEM((EK,), jnp.int32),
            pltpu.VMEM((EK,), jnp.int32),
            pltpu.VMEM((EK, D), jnp.float32),
            pltpu.VMEM((128, D), jnp.float32),
            pltpu.VMEM_SHARED((NPAD, D), jnp.float32),
            pltpu.SemaphoreType.DMA,
        ],
    )
    def agg(xs_hbm, src_hbm, dst_hbm, z128_hbm, out_hbm,
            sidx, didx, rows, obuf, acc, sem):
        c = lax.axis_index("c")
        s = lax.axis_index("s")

        pltpu.sync_copy(z128_hbm, obuf)

        def zero_b(i, _):
            pltpu.sync_copy(obuf, acc.at[pl.ds(s * APT + i * 128, 128)])
            return _
        lax.fori_loop(0, APT // 128, zero_b, None)
        plsc.subcore_barrier()

        def edge_chunk(k, _):
            base = c * EPC + s * EPT + k * EK
            pltpu.sync_copy(src_hbm.at[pl.ds(base, EK)], sidx)
            pltpu.sync_copy(dst_hbm.at[pl.ds(base, EK)], didx)
            pltpu.async_copy(xs_hbm.at[sidx], rows, sem).wait()
            pltpu.sync_copy(rows, acc.at[didx], add=True)
            return _
        lax.fori_loop(0, EPT // EK, edge_chunk, None)
        plsc.subcore_barrier()

        def out_b(i, _):
            rb = s * APT + i * 128
            pltpu.sync_copy(acc.at[pl.ds(rb, 128)], obuf)
            pltpu.sync_copy(obuf, out_hbm.at[pl.ds(c * NPAD + rb, 128)])
            return _
        lax.fori_loop(0, APT // 128, out_b, None)

    return agg


def _norm_from_deg(deg):
    return jnp.where(deg > 0.0, lax.rsqrt(jnp.maximum(deg, 1.0)), 0.0)


def _tc1_body(hsum_ref, degs_ref, w_ref, out_ref):
    nsrc = _norm_from_deg(jnp.sum(degs_ref[...], axis=1, keepdims=True))
    out_ref[...] = jnp.dot(hsum_ref[...], w_ref[...],
                           preferred_element_type=jnp.float32) * nsrc


def _tc2_body(a0_ref, a1_ref, degd_ref, degs_ref, b_ref, w_ref, out_ref):
    ndst = _norm_from_deg(jnp.sum(degd_ref[...], axis=1, keepdims=True))
    nsrc = _norm_from_deg(jnp.sum(degs_ref[...], axis=1, keepdims=True))
    h = jax.nn.relu((a0_ref[...] + a1_ref[...]) * ndst + b_ref[...])
    out_ref[...] = jnp.dot(h, w_ref[...],
                           preferred_element_type=jnp.float32) * nsrc


def _tc3_body(a0_ref, a1_ref, degd_ref, b_ref, wc_ref, bc_ref, out_ref,
              acc_ref):
    k = pl.program_id(0)

    @pl.when(k == 0)
    def _():
        acc_ref[...] = jnp.zeros_like(acc_ref)

    ndst = _norm_from_deg(jnp.sum(degd_ref[...], axis=1, keepdims=True))
    h = jax.nn.relu((a0_ref[...] + a1_ref[...]) * ndst + b_ref[...])
    acc_ref[...] += jnp.sum(h, axis=0, keepdims=True)

    @pl.when(k == pl.num_programs(0) - 1)
    def _():
        out_ref[...] = jnp.dot(acc_ref[...] * (1.0 / N), wc_ref[...],
                               preferred_element_type=jnp.float32) + bc_ref[...]


_SID_FLAT = np.repeat(np.arange(NPAD, dtype=np.int32) % NPC, L)
_Z128 = np.zeros((128, D), np.float32)


def kernel(ins, edge_index, emb, W1, b1, W2, b2, Wc, bc):
    src = edge_index[0].astype(jnp.int32)
    dst = edge_index[1].astype(jnp.int32)
    ins_flat = jnp.concatenate(
        [ins.reshape(-1).astype(jnp.int32),
         jnp.zeros(((NPAD - N) * L,), jnp.int32)])

    h_sum = _sc_emb(ins_flat, _SID_FLAT, emb, _Z128)
    deg = _sc_deg(src, dst).reshape(2 * NPAD, 16)
    deg_src = deg[:NPAD]
    deg_dst = deg[NPAD:]

    xs1 = pl.pallas_call(
        _tc1_body,
        grid=(NPAD // 512,),
        in_specs=[
            pl.BlockSpec((512, D), lambda i: (i, 0)),
            pl.BlockSpec((512, 16), lambda i: (i, 0)),
            pl.BlockSpec((D, H), lambda i: (0, 0)),
        ],
        out_specs=pl.BlockSpec((512, H), lambda i: (i, 0)),
        out_shape=jax.ShapeDtypeStruct((NPAD, H), jnp.float32),
    )(h_sum, deg_src, W1 * (1.0 / L))

    agg1 = _make_edge_agg()(xs1, src, dst, _Z128)

    xs2 = pl.pallas_call(
        _tc2_body,
        grid=(N // 400,),
        in_specs=[
            pl.BlockSpec((400, D), lambda i: (i, 0)),
            pl.BlockSpec((400, D), lambda i: (i, 0)),
            pl.BlockSpec((400, 16), lambda i: (i, 0)),
            pl.BlockSpec((400, 16), lambda i: (i, 0)),
            pl.BlockSpec((1, H), lambda i: (0, 0)),
            pl.BlockSpec((H, H), lambda i: (0, 0)),
        ],
        out_specs=pl.BlockSpec((400, H), lambda i: (i, 0)),
        out_shape=jax.ShapeDtypeStruct((N, H), jnp.float32),
    )(agg1[:N], agg1[NPAD:NPAD + N], deg_dst, deg_src, b1.reshape(1, H), W2)

    agg2 = _make_edge_agg()(xs2, src, dst, _Z128)

    out = pl.pallas_call(
        _tc3_body,
        grid=(N // 400,),
        in_specs=[
            pl.BlockSpec((400, H), lambda i: (i, 0)),
            pl.BlockSpec((400, H), lambda i: (i, 0)),
            pl.BlockSpec((400, 16), lambda i: (i, 0)),
            pl.BlockSpec((1, H), lambda i: (0, 0)),
            pl.BlockSpec((H, C), lambda i: (0, 0)),
            pl.BlockSpec((1, C), lambda i: (0, 0)),
        ],
        out_specs=pl.BlockSpec((1, C), lambda i: (0, 0)),
        out_shape=jax.ShapeDtypeStruct((1, C), jnp.float32),
        scratch_shapes=[pltpu.VMEM((1, H), jnp.float32)],
    )(agg2[:N], agg2[NPAD:NPAD + N], deg_dst, b2.reshape(1, H), Wc,
      bc.reshape(1, C))

    return out


# speedup vs baseline: 4.2342x; 1.2515x over previous
import functools

import jax
import jax.numpy as jnp
import numpy as np
from jax import lax
from jax.experimental import pallas as pl
from jax.experimental.pallas import tpu as pltpu
from jax.experimental.pallas import tpu_sc as plsc

N = 10000
E = 320000
V = 100000
L = 20
D = 128
H = 128
C = 10

NC = 2
NS = 16
NPAD = 10240
NPC = NPAD // NC
NPT = NPC // NS
EK = 80
EPC = E // NC
EPT = EPC // NS
APT = NPAD // NS

_mesh = plsc.VectorSubcoreMesh(core_axis_name="c", subcore_axis_name="s")


@functools.partial(
    pl.kernel,
    out_type=jax.ShapeDtypeStruct((NPAD, D), jnp.float32),
    mesh=_mesh,
    scratch_types=[
        pltpu.VMEM((EK,), jnp.int32),
        pltpu.VMEM((EK,), jnp.int32),
        pltpu.VMEM((EK, D), jnp.float32),
        pltpu.VMEM_SHARED((NPC, D), jnp.float32),
        pltpu.SemaphoreType.DMA,
    ],
)
def _sc_emb(ins_hbm, sid_hbm, emb_hbm, z128_hbm, hsum_hbm,
            gidx, sidv, rows, acc_h, sem):
    c = lax.axis_index("c")
    s = lax.axis_index("s")
    gnb = c * NPC + s * NPT

    pltpu.sync_copy(z128_hbm.at[pl.ds(0, EK)], rows)

    def zero_h(b, _):
        pltpu.sync_copy(rows, acc_h.at[pl.ds(s * NPT + b * EK, EK)])
        return _
    lax.fori_loop(0, NPT // EK, zero_h, None)
    plsc.subcore_barrier()

    def emb_chunk(k, _):
        base = gnb * L + k * EK
        pltpu.sync_copy(ins_hbm.at[pl.ds(base, EK)], gidx)
        pltpu.sync_copy(sid_hbm.at[pl.ds(base, EK)], sidv)
        pltpu.async_copy(emb_hbm.at[gidx], rows, sem).wait()
        pltpu.sync_copy(rows, acc_h.at[sidv], add=True)
        return _
    lax.fori_loop(0, (NPT * L) // EK, emb_chunk, None)
    plsc.subcore_barrier()

    def out_h(b, _):
        pltpu.sync_copy(acc_h.at[pl.ds(s * NPT + b * EK, EK)], rows)
        pltpu.sync_copy(rows, hsum_hbm.at[pl.ds(gnb + b * EK, EK)])
        return _
    lax.fori_loop(0, NPT // EK, out_h, None)


PN = NPC // 2
EPT_D = E // NS
ECH = 2000
RPT_D = PN // NS


@functools.partial(
    pl.kernel,
    out_type=jax.ShapeDtypeStruct((2 * NPAD * 16,), jnp.float32),
    mesh=_mesh,
    scratch_types=[
        pltpu.VMEM((ECH,), jnp.int32),
        pltpu.VMEM(((PN + 1) * 16,), jnp.float32),
        pltpu.VMEM((RPT_D * 16,), jnp.float32),
        pltpu.VMEM_SHARED((NS, PN * 16), jnp.float32),
    ],
    compiler_params=pltpu.CompilerParams(needs_layout_passes=False),
)
def _sc_deg(src_hbm, dst_hbm, deg_hbm, eidx, hist, out16, shp):
    c = lax.axis_index("c")
    s = lax.axis_index("s")
    lane = lax.iota(jnp.int32, 16)
    ones = jnp.full((16,), 1.0, jnp.float32)
    zeros = jnp.zeros((16,), jnp.float32)

    for p in range(NPAD // PN):
        def zero_row(r, _):
            hist[pl.ds(r * 16, 16)] = zeros
            return _
        lax.fori_loop(0, PN + 1, zero_row, None)

        def scatter(ref):
            def outer(i, _):
                pltpu.sync_copy(ref.at[pl.ds(s * EPT_D + i * ECH, ECH)], eidx)

                def inner(j, __):
                    ids = eidx[pl.ds(j * 16, 16)]
                    rel = ids - p * PN
                    ok = (rel >= 0) & (rel < PN)
                    addr = jnp.where(ok, rel, PN) * 16 + lane
                    v = plsc.load_gather(hist, [addr])
                    plsc.store_scatter(hist, [addr], v + ones)
                    return __
                lax.fori_loop(0, ECH // 16, inner, None)
                return _
            lax.fori_loop(0, EPT_D // ECH, outer, None)

        @pl.when(c == 0)
        def _():
            scatter(src_hbm)

        @pl.when(c == 1)
        def _():
            scatter(dst_hbm)

        pltpu.sync_copy(hist.at[pl.ds(0, PN * 16)], shp.at[s])
        plsc.subcore_barrier()

        def fetch(t, _):
            pltpu.sync_copy(shp.at[t, pl.ds(s * RPT_D * 16, RPT_D * 16)],
                            hist.at[pl.ds(t * RPT_D * 16, RPT_D * 16)])
            return _
        lax.fori_loop(0, NS, fetch, None)

        def reduce_row(r, _):
            v = hist[pl.ds(r * 16, 16)]
            for t in range(1, NS):
                v = v + hist[pl.ds((t * RPT_D + r) * 16, 16)]
            out16[pl.ds(r * 16, 16)] = v
            return _
        lax.fori_loop(0, RPT_D, reduce_row, None)

        pltpu.sync_copy(
            out16,
            deg_hbm.at[pl.ds((c * NPAD + p * PN + s * RPT_D) * 16,
                             RPT_D * 16)])
        plsc.subcore_barrier()


def _make_edge_agg():
    @functools.partial(
        pl.kernel,
        out_type=jax.ShapeDtypeStruct((2 * NPAD, D), jnp.float32),
        mesh=_mesh,
        scratch_types=[
            pltpu.VMEM((EK,), jnp.int32),
            pltpu.VMEM((EK,), jnp.int32),
            pltpu.VMEM((EK, D), jnp.float32),
            pltpu.VMEM((EK,), jnp.int32),
            pltpu.VMEM((EK,), jnp.int32),
            pltpu.VMEM((EK, D), jnp.float32),
            pltpu.VMEM((128, D), jnp.float32),
            pltpu.VMEM_SHARED((NPAD, D), jnp.float32),
            pltpu.SemaphoreType.DMA,
            pltpu.SemaphoreType.DMA,
        ],
    )
    def agg(xs_hbm, src_hbm, dst_hbm, z128_hbm, out_hbm,
            sidxa, didxa, rowsa, sidxb, didxb, rowsb, obuf, acc,
            sema, semb):
        c = lax.axis_index("c")
        s = lax.axis_index("s")

        pltpu.sync_copy(z128_hbm, obuf)

        def zero_b(i, _):
            pltpu.sync_copy(obuf, acc.at[pl.ds(s * APT + i * 128, 128)])
            return _
        lax.fori_loop(0, APT // 128, zero_b, None)
        plsc.subcore_barrier()

        ebase = c * EPC + s * EPT
        NCHK = EPT // EK

        def start(k, sidx, didx, rows, sem):
            base = ebase + k * EK
            pltpu.sync_copy(src_hbm.at[pl.ds(base, EK)], sidx)
            pltpu.sync_copy(dst_hbm.at[pl.ds(base, EK)], didx)
            return pltpu.async_copy(xs_hbm.at[sidx], rows, sem)

        def drain(didx, rows, sem):
            pltpu.make_async_copy(xs_hbm.at[pl.ds(0, EK)], rows, sem).wait()
            pltpu.sync_copy(rows, acc.at[didx], add=True)

        start(0, sidxa, didxa, rowsa, sema)

        def pair(g, _):
            start(2 * g + 1, sidxb, didxb, rowsb, semb)
            drain(didxa, rowsa, sema)
            start(2 * g + 2, sidxa, didxa, rowsa, sema)
            drain(didxb, rowsb, semb)
            return _
        lax.fori_loop(0, (NCHK - 1) // 2, pair, None)
        drain(didxa, rowsa, sema)
        plsc.subcore_barrier()

        def out_b(i, _):
            rb = s * APT + i * 128
            pltpu.sync_copy(acc.at[pl.ds(rb, 128)], obuf)
            pltpu.sync_copy(obuf, out_hbm.at[pl.ds(c * NPAD + rb, 128)])
            return _
        lax.fori_loop(0, APT // 128, out_b, None)

    return agg


def _norm_from_deg(deg):
    return jnp.where(deg > 0.0, lax.rsqrt(jnp.maximum(deg, 1.0)), 0.0)


def _tc1_body(hsum_ref, degs_ref, w_ref, out_ref):
    nsrc = _norm_from_deg(jnp.sum(degs_ref[...], axis=1, keepdims=True))
    out_ref[...] = jnp.dot(hsum_ref[...], w_ref[...],
                           preferred_element_type=jnp.float32) * nsrc


def _tc2_body(a0_ref, a1_ref, degd_ref, degs_ref, b_ref, w_ref, out_ref):
    ndst = _norm_from_deg(jnp.sum(degd_ref[...], axis=1, keepdims=True))
    nsrc = _norm_from_deg(jnp.sum(degs_ref[...], axis=1, keepdims=True))
    h = jax.nn.relu((a0_ref[...] + a1_ref[...]) * ndst + b_ref[...])
    out_ref[...] = jnp.dot(h, w_ref[...],
                           preferred_element_type=jnp.float32) * nsrc


def _tc3_body(a0_ref, a1_ref, degd_ref, b_ref, wc_ref, bc_ref, out_ref,
              acc_ref):
    k = pl.program_id(0)

    @pl.when(k == 0)
    def _():
        acc_ref[...] = jnp.zeros_like(acc_ref)

    ndst = _norm_from_deg(jnp.sum(degd_ref[...], axis=1, keepdims=True))
    h = jax.nn.relu((a0_ref[...] + a1_ref[...]) * ndst + b_ref[...])
    acc_ref[...] += jnp.sum(h, axis=0, keepdims=True)

    @pl.when(k == pl.num_programs(0) - 1)
    def _():
        out_ref[...] = jnp.dot(acc_ref[...] * (1.0 / N), wc_ref[...],
                               preferred_element_type=jnp.float32) + bc_ref[...]


_SID_FLAT = np.repeat(np.arange(NPAD, dtype=np.int32) % NPC, L)
_Z128 = np.zeros((128, D), np.float32)


def kernel(ins, edge_index, emb, W1, b1, W2, b2, Wc, bc):
    src = edge_index[0].astype(jnp.int32)
    dst = edge_index[1].astype(jnp.int32)
    ins_flat = jnp.concatenate(
        [ins.reshape(-1).astype(jnp.int32),
         jnp.zeros(((NPAD - N) * L,), jnp.int32)])

    h_sum = _sc_emb(ins_flat, _SID_FLAT, emb, _Z128)
    deg = _sc_deg(src, dst).reshape(2 * NPAD, 16)
    deg_src = deg[:NPAD]
    deg_dst = deg[NPAD:]

    xs1 = pl.pallas_call(
        _tc1_body,
        grid=(NPAD // 512,),
        in_specs=[
            pl.BlockSpec((512, D), lambda i: (i, 0)),
            pl.BlockSpec((512, 16), lambda i: (i, 0)),
            pl.BlockSpec((D, H), lambda i: (0, 0)),
        ],
        out_specs=pl.BlockSpec((512, H), lambda i: (i, 0)),
        out_shape=jax.ShapeDtypeStruct((NPAD, H), jnp.float32),
    )(h_sum, deg_src, W1 * (1.0 / L))

    agg1 = _make_edge_agg()(xs1, src, dst, _Z128)

    xs2 = pl.pallas_call(
        _tc2_body,
        grid=(N // 400,),
        in_specs=[
            pl.BlockSpec((400, D), lambda i: (i, 0)),
            pl.BlockSpec((400, D), lambda i: (i, 0)),
            pl.BlockSpec((400, 16), lambda i: (i, 0)),
            pl.BlockSpec((400, 16), lambda i: (i, 0)),
            pl.BlockSpec((1, H), lambda i: (0, 0)),
            pl.BlockSpec((H, H), lambda i: (0, 0)),
        ],
        out_specs=pl.BlockSpec((400, H), lambda i: (i, 0)),
        out_shape=jax.ShapeDtypeStruct((N, H), jnp.float32),
    )(agg1[:N], agg1[NPAD:NPAD + N], deg_dst, deg_src, b1.reshape(1, H), W2)

    agg2 = _make_edge_agg()(xs2, src, dst, _Z128)

    out = pl.pallas_call(
        _tc3_body,
        grid=(N // 400,),
        in_specs=[
            pl.BlockSpec((400, H), lambda i: (i, 0)),
            pl.BlockSpec((400, H), lambda i: (i, 0)),
            pl.BlockSpec((400, 16), lambda i: (i, 0)),
            pl.BlockSpec((1, H), lambda i: (0, 0)),
            pl.BlockSpec((H, C), lambda i: (0, 0)),
            pl.BlockSpec((1, C), lambda i: (0, 0)),
        ],
        out_specs=pl.BlockSpec((1, C), lambda i: (0, 0)),
        out_shape=jax.ShapeDtypeStruct((1, C), jnp.float32),
        scratch_shapes=[pltpu.VMEM((1, H), jnp.float32)],
    )(agg2[:N], agg2[NPAD:NPAD + N], deg_dst, b2.reshape(1, H), Wc,
      bc.reshape(1, C))

    return out


# speedup vs baseline: 4.7242x; 1.1157x over previous
import functools

import jax
import jax.numpy as jnp
import numpy as np
from jax import lax
from jax.experimental import pallas as pl
from jax.experimental.pallas import tpu as pltpu
from jax.experimental.pallas import tpu_sc as plsc

N = 10000
E = 320000
V = 100000
L = 20
D = 128
H = 128
C = 10

NC = 2
NS = 16
NPAD = 10240
NPC = NPAD // NC
NPT = NPC // NS
EK = 80
EPC = E // NC
EPT = EPC // NS
APT = NPAD // NS

_mesh = plsc.VectorSubcoreMesh(core_axis_name="c", subcore_axis_name="s")


@functools.partial(
    pl.kernel,
    out_type=jax.ShapeDtypeStruct((NPAD, D), jnp.float32),
    mesh=_mesh,
    scratch_types=[
        pltpu.VMEM((EK,), jnp.int32),
        pltpu.VMEM((EK,), jnp.int32),
        pltpu.VMEM((EK, D), jnp.float32),
        pltpu.VMEM((EK,), jnp.int32),
        pltpu.VMEM((EK,), jnp.int32),
        pltpu.VMEM((EK, D), jnp.float32),
        pltpu.VMEM_SHARED((NPC, D), jnp.float32),
        pltpu.SemaphoreType.DMA,
        pltpu.SemaphoreType.DMA,
    ],
)
def _sc_emb(ins_hbm, sid_hbm, emb_hbm, z128_hbm, hsum_hbm,
            gidxa, sidva, rowsa, gidxb, sidvb, rowsb, acc_h, sema, semb):
    c = lax.axis_index("c")
    s = lax.axis_index("s")
    gnb = c * NPC + s * NPT

    pltpu.sync_copy(z128_hbm.at[pl.ds(0, EK)], rowsa)

    def zero_h(b, _):
        pltpu.sync_copy(rowsa, acc_h.at[pl.ds(s * NPT + b * EK, EK)])
        return _
    lax.fori_loop(0, NPT // EK, zero_h, None)
    plsc.subcore_barrier()

    NCHK = (NPT * L) // EK

    def start(k, gidx, sidv, rows, sem):
        base = gnb * L + k * EK
        pltpu.sync_copy(ins_hbm.at[pl.ds(base, EK)], gidx)
        pltpu.sync_copy(sid_hbm.at[pl.ds(base, EK)], sidv)
        return pltpu.async_copy(emb_hbm.at[gidx], rows, sem)

    def drain(sidv, rows, sem):
        pltpu.make_async_copy(emb_hbm.at[pl.ds(0, EK)], rows, sem).wait()
        pltpu.sync_copy(rows, acc_h.at[sidv], add=True)

    start(0, gidxa, sidva, rowsa, sema)

    def pair(g, _):
        start(2 * g + 1, gidxb, sidvb, rowsb, semb)
        drain(sidva, rowsa, sema)
        start(2 * g + 2, gidxa, sidva, rowsa, sema)
        drain(sidvb, rowsb, semb)
        return _
    lax.fori_loop(0, (NCHK - 1) // 2, pair, None)
    if NCHK % 2 == 1:
        drain(sidva, rowsa, sema)
    else:
        start(NCHK - 1, gidxb, sidvb, rowsb, semb)
        drain(sidva, rowsa, sema)
        drain(sidvb, rowsb, semb)
    plsc.subcore_barrier()

    def out_h(b, _):
        pltpu.sync_copy(acc_h.at[pl.ds(s * NPT + b * EK, EK)], rowsa)
        pltpu.sync_copy(rowsa, hsum_hbm.at[pl.ds(gnb + b * EK, EK)])
        return _
    lax.fori_loop(0, NPT // EK, out_h, None)


PN = NPC // 2
EPT_D = E // NS
ECH = 2000
RPT_D = PN // NS


@functools.partial(
    pl.kernel,
    out_type=jax.ShapeDtypeStruct((2 * NPAD * 16,), jnp.float32),
    mesh=_mesh,
    scratch_types=[
        pltpu.VMEM((ECH,), jnp.int32),
        pltpu.VMEM(((PN + 1) * 16,), jnp.float32),
        pltpu.VMEM((RPT_D * 16,), jnp.float32),
        pltpu.VMEM_SHARED((NS, PN * 16), jnp.float32),
    ],
    compiler_params=pltpu.CompilerParams(needs_layout_passes=False),
)
def _sc_deg(src_hbm, dst_hbm, deg_hbm, eidx, hist, out16, shp):
    c = lax.axis_index("c")
    s = lax.axis_index("s")
    lane = lax.iota(jnp.int32, 16)
    ones = jnp.full((16,), 1.0, jnp.float32)
    zeros = jnp.zeros((16,), jnp.float32)

    for p in range(NPAD // PN):
        def zero_row(r, _):
            hist[pl.ds(r * 16, 16)] = zeros
            return _
        lax.fori_loop(0, PN + 1, zero_row, None)

        def scatter(ref):
            def outer(i, _):
                pltpu.sync_copy(ref.at[pl.ds(s * EPT_D + i * ECH, ECH)], eidx)

                def inner(j, __):
                    ids = eidx[pl.ds(j * 16, 16)]
                    rel = ids - p * PN
                    ok = (rel >= 0) & (rel < PN)
                    addr = jnp.where(ok, rel, PN) * 16 + lane
                    v = plsc.load_gather(hist, [addr])
                    plsc.store_scatter(hist, [addr], v + ones)
                    return __
                lax.fori_loop(0, ECH // 16, inner, None)
                return _
            lax.fori_loop(0, EPT_D // ECH, outer, None)

        @pl.when(c == 0)
        def _():
            scatter(src_hbm)

        @pl.when(c == 1)
        def _():
            scatter(dst_hbm)

        pltpu.sync_copy(hist.at[pl.ds(0, PN * 16)], shp.at[s])
        plsc.subcore_barrier()

        def fetch(t, _):
            pltpu.sync_copy(shp.at[t, pl.ds(s * RPT_D * 16, RPT_D * 16)],
                            hist.at[pl.ds(t * RPT_D * 16, RPT_D * 16)])
            return _
        lax.fori_loop(0, NS, fetch, None)

        def reduce_row(r, _):
            v = hist[pl.ds(r * 16, 16)]
            for t in range(1, NS):
                v = v + hist[pl.ds((t * RPT_D + r) * 16, 16)]
            out16[pl.ds(r * 16, 16)] = v
            return _
        lax.fori_loop(0, RPT_D, reduce_row, None)

        pltpu.sync_copy(
            out16,
            deg_hbm.at[pl.ds((c * NPAD + p * PN + s * RPT_D) * 16,
                             RPT_D * 16)])
        plsc.subcore_barrier()


def _make_edge_agg():
    @functools.partial(
        pl.kernel,
        out_type=jax.ShapeDtypeStruct((2 * NPAD, D), jnp.float32),
        mesh=_mesh,
        scratch_types=[
            pltpu.VMEM((EK,), jnp.int32),
            pltpu.VMEM((EK,), jnp.int32),
            pltpu.VMEM((EK, D), jnp.float32),
            pltpu.VMEM((EK,), jnp.int32),
            pltpu.VMEM((EK,), jnp.int32),
            pltpu.VMEM((EK, D), jnp.float32),
            pltpu.VMEM((128, D), jnp.float32),
            pltpu.VMEM_SHARED((NPAD, D), jnp.float32),
            pltpu.SemaphoreType.DMA,
            pltpu.SemaphoreType.DMA,
        ],
    )
    def agg(xs_hbm, src_hbm, dst_hbm, z128_hbm, out_hbm,
            sidxa, didxa, rowsa, sidxb, didxb, rowsb, obuf, acc,
            sema, semb):
        c = lax.axis_index("c")
        s = lax.axis_index("s")

        pltpu.sync_copy(z128_hbm, obuf)

        def zero_b(i, _):
            pltpu.sync_copy(obuf, acc.at[pl.ds(s * APT + i * 128, 128)])
            return _
        lax.fori_loop(0, APT // 128, zero_b, None)
        plsc.subcore_barrier()

        ebase = c * EPC + s * EPT
        NCHK = EPT // EK

        def start(k, sidx, didx, rows, sem):
            base = ebase + k * EK
            pltpu.sync_copy(src_hbm.at[pl.ds(base, EK)], sidx)
            pltpu.sync_copy(dst_hbm.at[pl.ds(base, EK)], didx)
            return pltpu.async_copy(xs_hbm.at[sidx], rows, sem)

        def drain(didx, rows, sem):
            pltpu.make_async_copy(xs_hbm.at[pl.ds(0, EK)], rows, sem).wait()
            pltpu.sync_copy(rows, acc.at[didx], add=True)

        start(0, sidxa, didxa, rowsa, sema)

        def pair(g, _):
            start(2 * g + 1, sidxb, didxb, rowsb, semb)
            drain(didxa, rowsa, sema)
            start(2 * g + 2, sidxa, didxa, rowsa, sema)
            drain(didxb, rowsb, semb)
            return _
        lax.fori_loop(0, (NCHK - 1) // 2, pair, None)
        drain(didxa, rowsa, sema)
        plsc.subcore_barrier()

        def out_b(i, _):
            rb = s * APT + i * 128
            pltpu.sync_copy(acc.at[pl.ds(rb, 128)], obuf)
            pltpu.sync_copy(obuf, out_hbm.at[pl.ds(c * NPAD + rb, 128)])
            return _
        lax.fori_loop(0, APT // 128, out_b, None)

    return agg


def _norm_from_deg(deg):
    return jnp.where(deg > 0.0, lax.rsqrt(jnp.maximum(deg, 1.0)), 0.0)


def _tc1_body(hsum_ref, degs_ref, w_ref, out_ref):
    nsrc = _norm_from_deg(jnp.sum(degs_ref[...], axis=1, keepdims=True))
    out_ref[...] = jnp.dot(hsum_ref[...], w_ref[...],
                           preferred_element_type=jnp.float32) * nsrc


def _tc2_body(a0_ref, a1_ref, degd_ref, degs_ref, b_ref, w_ref, out_ref):
    ndst = _norm_from_deg(jnp.sum(degd_ref[...], axis=1, keepdims=True))
    nsrc = _norm_from_deg(jnp.sum(degs_ref[...], axis=1, keepdims=True))
    h = jax.nn.relu((a0_ref[...] + a1_ref[...]) * ndst + b_ref[...])
    out_ref[...] = jnp.dot(h, w_ref[...],
                           preferred_element_type=jnp.float32) * nsrc


def _tc3_body(a0_ref, a1_ref, degd_ref, b_ref, wc_ref, bc_ref, out_ref,
              acc_ref):
    k = pl.program_id(0)

    @pl.when(k == 0)
    def _():
        acc_ref[...] = jnp.zeros_like(acc_ref)

    ndst = _norm_from_deg(jnp.sum(degd_ref[...], axis=1, keepdims=True))
    h = jax.nn.relu((a0_ref[...] + a1_ref[...]) * ndst + b_ref[...])
    acc_ref[...] += jnp.sum(h, axis=0, keepdims=True)

    @pl.when(k == pl.num_programs(0) - 1)
    def _():
        out_ref[...] = jnp.dot(acc_ref[...] * (1.0 / N), wc_ref[...],
                               preferred_element_type=jnp.float32) + bc_ref[...]


_SID_FLAT = np.repeat(np.arange(NPAD, dtype=np.int32) % NPC, L)
_Z128 = np.zeros((128, D), np.float32)


def kernel(ins, edge_index, emb, W1, b1, W2, b2, Wc, bc):
    src = edge_index[0].astype(jnp.int32)
    dst = edge_index[1].astype(jnp.int32)
    ins_flat = jnp.concatenate(
        [ins.reshape(-1).astype(jnp.int32),
         jnp.zeros(((NPAD - N) * L,), jnp.int32)])

    h_sum = _sc_emb(ins_flat, _SID_FLAT, emb, _Z128)
    deg = _sc_deg(src, dst).reshape(2 * NPAD, 16)
    deg_src = deg[:NPAD]
    deg_dst = deg[NPAD:]

    xs1 = pl.pallas_call(
        _tc1_body,
        grid=(NPAD // 512,),
        in_specs=[
            pl.BlockSpec((512, D), lambda i: (i, 0)),
            pl.BlockSpec((512, 16), lambda i: (i, 0)),
            pl.BlockSpec((D, H), lambda i: (0, 0)),
        ],
        out_specs=pl.BlockSpec((512, H), lambda i: (i, 0)),
        out_shape=jax.ShapeDtypeStruct((NPAD, H), jnp.float32),
    )(h_sum, deg_src, W1 * (1.0 / L))

    agg1 = _make_edge_agg()(xs1, src, dst, _Z128)

    xs2 = pl.pallas_call(
        _tc2_body,
        grid=(N // 400,),
        in_specs=[
            pl.BlockSpec((400, D), lambda i: (i, 0)),
            pl.BlockSpec((400, D), lambda i: (i, 0)),
            pl.BlockSpec((400, 16), lambda i: (i, 0)),
            pl.BlockSpec((400, 16), lambda i: (i, 0)),
            pl.BlockSpec((1, H), lambda i: (0, 0)),
            pl.BlockSpec((H, H), lambda i: (0, 0)),
        ],
        out_specs=pl.BlockSpec((400, H), lambda i: (i, 0)),
        out_shape=jax.ShapeDtypeStruct((N, H), jnp.float32),
    )(agg1[:N], agg1[NPAD:NPAD + N], deg_dst, deg_src, b1.reshape(1, H), W2)

    agg2 = _make_edge_agg()(xs2, src, dst, _Z128)

    out = pl.pallas_call(
        _tc3_body,
        grid=(N // 400,),
        in_specs=[
            pl.BlockSpec((400, H), lambda i: (i, 0)),
            pl.BlockSpec((400, H), lambda i: (i, 0)),
            pl.BlockSpec((400, 16), lambda i: (i, 0)),
            pl.BlockSpec((1, H), lambda i: (0, 0)),
            pl.BlockSpec((H, C), lambda i: (0, 0)),
            pl.BlockSpec((1, C), lambda i: (0, 0)),
        ],
        out_specs=pl.BlockSpec((1, C), lambda i: (0, 0)),
        out_shape=jax.ShapeDtypeStruct((1, C), jnp.float32),
        scratch_shapes=[pltpu.VMEM((1, H), jnp.float32)],
    )(agg2[:N], agg2[NPAD:NPAD + N], deg_dst, b2.reshape(1, H), Wc,
      bc.reshape(1, C))

    return out
